# baseline (speedup 1.0000x reference)
import dataclasses
import functools

import jax
import jax.numpy as jnp
from jax import lax
from jax.experimental import pallas as pl
from jax.experimental.pallas import tpu as pltpu
from jax.experimental.pallas import tpu_sc as plsc

NC, NS = 2, 16
ACC_BIG = 50176
ACC_SMALL = 3584
ZCH = 112
N_ITEMS = 50000
N_USERS = 50000
N_ATTRS = 2000
BN = 1000


def _acc_rows(nd):
    return ACC_SMALL if nd <= ACC_SMALL - 1 else ACC_BIG


_SC_PARAMS = dataclasses.replace(pltpu.CompilerParams(),
                                 use_tc_tiling_on_sc=False)


@functools.partial(jax.jit, static_argnums=(2, 3, 4))
def _sc_agg(tables, edges, acc_rows, G, K):
    src_h, dst_h = edges
    nz = acc_rows // NS // ZCH
    rows_sub = acc_rows // NS
    mesh = plsc.VectorSubcoreMesh(core_axis_name="c", subcore_axis_name="s")

    @functools.partial(
        pl.kernel,
        out_type=jax.ShapeDtypeStruct((NC, 2, acc_rows, 64), jnp.bfloat16),
        mesh=mesh,
        scratch_types=[
            pltpu.VMEM((K, 128), jnp.int32),
            pltpu.VMEM((K, 128), jnp.int32),
            pltpu.VMEM((128, 64), jnp.bfloat16),
            pltpu.VMEM_SHARED((acc_rows, 64), jnp.bfloat16),
        ],
        compiler_params=_SC_PARAMS,
    )
    def body(x0, x1, zb, s_hbm, d_hbm, out_hbm,
             sidx, didx, rows, acc):
        c = lax.axis_index("c")
        s = lax.axis_index("s")
        r0 = s * rows_sub
        for q, xq in enumerate((x0, x1)):
            for j in range(nz):
                pltpu.sync_copy(zb.at[pl.ds(j * ZCH, ZCH)],
                                acc.at[pl.ds(r0 + j * ZCH, ZCH)])
            plsc.subcore_barrier()

            @pl.loop(0, G)
            def _(g):
                pltpu.sync_copy(s_hbm.at[c, s, g], sidx)
                pltpu.sync_copy(d_hbm.at[c, s, g], didx)
                for k in range(K):
                    pltpu.sync_copy(xq.at[sidx.at[k]], rows)
                    pltpu.sync_copy(rows, acc.at[didx.at[k]], add=True)

            plsc.subcore_barrier()
            for j in range(nz):
                r = r0 + j * ZCH
                pltpu.sync_copy(acc.at[pl.ds(r, ZCH)],
                                out_hbm.at[c, q, pl.ds(r, ZCH)])
            plsc.subcore_barrier()

    zb = jnp.zeros((rows_sub, 64), jnp.bfloat16)
    return body(tables[0], tables[1], zb, src_h, dst_h)


@functools.partial(jax.jit, static_argnums=(1, 2, 3))
def _sc_count(dst_h, acc_rows, G, K):
    nz = acc_rows // NS // ZCH
    rows_sub = acc_rows // NS
    mesh = plsc.VectorSubcoreMesh(core_axis_name="c", subcore_axis_name="s")

    @functools.partial(
        pl.kernel,
        out_type=jax.ShapeDtypeStruct((NC, acc_rows, 16), jnp.float32),
        mesh=mesh,
        scratch_types=[
            pltpu.VMEM((K, 128), jnp.int32),
            pltpu.VMEM((128, 16), jnp.float32),
            pltpu.VMEM_SHARED((acc_rows, 16), jnp.float32),
        ],
        compiler_params=_SC_PARAMS,
    )
    def body(ones_hbm, zb, d_hbm, out_hbm, didx, ones, acc):
        c = lax.axis_index("c")
        s = lax.axis_index("s")
        pltpu.sync_copy(ones_hbm, ones)
        r0 = s * rows_sub
        for j in range(nz):
            pltpu.sync_copy(zb.at[pl.ds(j * ZCH, ZCH)],
                            acc.at[pl.ds(r0 + j * ZCH, ZCH)])
        plsc.subcore_barrier()

        @pl.loop(0, G)
        def _(g):
            pltpu.sync_copy(d_hbm.at[c, s, g], didx)
            for k in range(K):
                pltpu.sync_copy(ones, acc.at[didx.at[k]], add=True)

        plsc.subcore_barrier()
        for j in range(nz):
            r = r0 + j * ZCH
            pltpu.sync_copy(acc.at[pl.ds(r, ZCH)], out_hbm.at[c, pl.ds(r, ZCH)])
        plsc.subcore_barrier()

    ones = jnp.ones((128, 16), jnp.float32)
    zb = jnp.zeros((rows_sub, 16), jnp.float32)
    return body(ones, zb, dst_h)


def _prep_edges(ei, nd_dummy, G, K):
    e = ei.shape[1]
    epad = NC * NS * G * K * 128
    src = jnp.concatenate(
        [ei[0].astype(jnp.int32), jnp.zeros((epad - e,), jnp.int32)])
    dst = jnp.concatenate(
        [ei[1].astype(jnp.int32), jnp.full((epad - e,), nd_dummy, jnp.int32)])
    return (src.reshape(NC, NS, G, K, 128), dst.reshape(NC, NS, G, K, 128))


def _edge_plan(e):
    s = -(-e // (NC * NS * 128))
    if e >= 100000:
        s = -(-s // 8) * 8
        k = 8
    else:
        s += s % 2
        k = next(d for d in (8, 7, 6, 5, 4, 3, 2) if s % d == 0)
    return s // k, k


def _combine_body(p1, c1, p2, c2, xd, w1, w2, wr1, wr2, b1, b2, out, *tbl,
                  relu):
    inv1 = 1.0 / jnp.maximum(c1[0, :, 0] + c1[1, :, 0], 1.0)
    inv2 = 1.0 / jnp.maximum(c2[0, :, 0] + c2[1, :, 0], 1.0)
    acc = jnp.dot(xd[...], wr1[...] + wr2[...],
                  preferred_element_type=jnp.float32)
    acc += (b1[...] + b2[...])
    for h in range(2):
        m1 = (p1[0, h].astype(jnp.float32)
              + p1[1, h].astype(jnp.float32)) * inv1[:, None]
        m2 = (p2[0, h].astype(jnp.float32)
              + p2[1, h].astype(jnp.float32)) * inv2[:, None]
        acc += jnp.dot(m1, w1[h * 64:(h + 1) * 64, :],
                       preferred_element_type=jnp.float32)
        acc += jnp.dot(m2, w2[h * 64:(h + 1) * 64, :],
                       preferred_element_type=jnp.float32)
    acc *= 0.5
    o = jnp.maximum(acc, 0.0) if relu else acc
    out[...] = o
    for h in range(len(tbl)):
        tbl[h][...] = o[:, h * 64:(h + 1) * 64].astype(jnp.bfloat16)


@functools.partial(jax.jit, static_argnums=(11, 12))
def _tc_combine(p1, c1, p2, c2, xd, w1, w2, wr1, wr2, b1, b2, n, want_tbl):
    grid = (n // BN,)
    in_specs = [
        pl.BlockSpec((NC, 2, BN, 64), lambda i: (0, 0, i, 0)),
        pl.BlockSpec((NC, BN, 16), lambda i: (0, i, 0)),
        pl.BlockSpec((NC, 2, BN, 64), lambda i: (0, 0, i, 0)),
        pl.BlockSpec((NC, BN, 16), lambda i: (0, i, 0)),
        pl.BlockSpec((BN, 128), lambda i: (i, 0)),
        pl.BlockSpec((128, 128), lambda i: (0, 0)),
        pl.BlockSpec((128, 128), lambda i: (0, 0)),
        pl.BlockSpec((128, 128), lambda i: (0, 0)),
        pl.BlockSpec((128, 128), lambda i: (0, 0)),
        pl.BlockSpec((1, 128), lambda i: (0, 0)),
        pl.BlockSpec((1, 128), lambda i: (0, 0)),
    ]
    out_shape = [jax.ShapeDtypeStruct((n, 128), jnp.float32)]
    out_specs = [pl.BlockSpec((BN, 128), lambda i: (i, 0))]
    if want_tbl:
        out_shape += [jax.ShapeDtypeStruct((n, 64), jnp.bfloat16)] * 2
        out_specs += [pl.BlockSpec((BN, 64), lambda i: (i, 0))] * 2
    fn = pl.pallas_call(
        functools.partial(_combine_body, relu=True),
        grid=grid, in_specs=in_specs, out_specs=out_specs,
        out_shape=out_shape)
    res = fn(p1, c1, p2, c2, xd, w1, w2, wr1, wr2,
             b1.reshape(1, 128), b2.reshape(1, 128))
    if want_tbl:
        return res[0], res[1:]
    return res[0]


def _proj_body(x, w, b, out, *tbl):
    o = jnp.dot(x[...], w[...], preferred_element_type=jnp.float32) + b[...]
    out[...] = o
    for h in range(2):
        tbl[h][...] = o[:, h * 64:(h + 1) * 64].astype(jnp.bfloat16)


@jax.jit
def _tc_proj(x, w, b):
    n = x.shape[0]
    fn = pl.pallas_call(
        _proj_body,
        grid=(n // BN,),
        in_specs=[pl.BlockSpec((BN, 128), lambda i: (i, 0)),
                  pl.BlockSpec((128, 128), lambda i: (0, 0)),
                  pl.BlockSpec((1, 128), lambda i: (0, 0))],
        out_specs=[pl.BlockSpec((BN, 128), lambda i: (i, 0))]
        + [pl.BlockSpec((BN, 64), lambda i: (i, 0))] * 2,
        out_shape=[jax.ShapeDtypeStruct((n, 128), jnp.float32)]
        + [jax.ShapeDtypeStruct((n, 64), jnp.bfloat16)] * 2,
    )
    res = fn(x, w, b.reshape(1, 128))
    return res[0], res[1:]


def _fuse_body(apad, hui, hii, hia, out, wout):
    a = apad[...]
    m = jnp.max(a, axis=1, keepdims=True)
    e = jnp.exp(a - m)
    w = e / jnp.sum(e, axis=1, keepdims=True)
    out[...] = (hui[...] * w[0:1, 0:1] + hii[...] * w[0:1, 1:2]
                + hia[...] * w[0:1, 2:3])
    wout[...] = w


@jax.jit
def _tc_fuse(alpha, hui, hii, hia):
    n = hui.shape[0]
    apad = jnp.concatenate(
        [alpha, jnp.full((125,), -1e30, jnp.float32)]).reshape(1, 128)
    fn = pl.pallas_call(
        _fuse_body,
        grid=(n // BN,),
        in_specs=[pl.BlockSpec((1, 128), lambda i: (0, 0))]
        + [pl.BlockSpec((BN, 128), lambda i: (i, 0))] * 3,
        out_specs=[pl.BlockSpec((BN, 128), lambda i: (i, 0)),
                   pl.BlockSpec((1, 128), lambda i: (0, 0))],
        out_shape=[jax.ShapeDtypeStruct((n, 128), jnp.float32),
                   jax.ShapeDtypeStruct((1, 128), jnp.float32)],
    )
    out, wout = fn(apad, hui, hii, hia)
    return out, wout[0, :3]


def _split_cols(x):
    return tuple(x[:, h * 64:(h + 1) * 64].astype(jnp.bfloat16)
                 for h in range(2))


def kernel(x_item, ei_rates, ei_rev_rates, ei_user_self, ei_item_self,
           ei_sim, ei_has, ei_rev_has, ei_attr_self, params):
    p = params
    eis = {
        "rates": (ei_rates, N_ITEMS),
        "rev_rates": (ei_rev_rates, N_USERS),
        "user_self": (ei_user_self, N_USERS),
        "item_self": (ei_item_self, N_ITEMS),
        "sim": (ei_sim, N_ITEMS),
        "has": (ei_has, N_ATTRS),
        "rev_has": (ei_rev_has, N_ITEMS),
        "attr_self": (ei_attr_self, N_ATTRS),
    }
    prep, cnts, plans = {}, {}, {}
    for name, (ei, nd) in eis.items():
        g, k = _edge_plan(ei.shape[1])
        ar = _acc_rows(nd)
        plans[name] = (ar, g, k)
        prep[name] = _prep_edges(ei, nd, g, k)
        cnts[name] = _sc_count(prep[name][1], ar, g, k)

    def agg(name, tables):
        ar, g, k = plans[name]
        return _sc_agg(tables, prep[name], ar, g, k)

    def combine(pre1, a1, c1, pre2, a2, c2, xd, n, want_tbl):
        return _tc_combine(a1, c1, a2, c2, xd,
                           p[pre1 + "_Wl"], p[pre2 + "_Wl"],
                           p[pre1 + "_Wr"], p[pre2 + "_Wr"],
                           p[pre1 + "_bl"], p[pre2 + "_bl"], n, want_tbl)

    xi, xi_tbl = _tc_proj(x_item, p["lin_item_W"], p["lin_item_b"])
    xu = p["user_emb"]
    xu_tbl = _split_cols(xu)
    xa = p["attr_emb"]
    xa_tbl = _split_cols(xa)

    a_rates = agg("rates", xu_tbl)
    a_item_self = agg("item_self", xi_tbl)
    a_rev_rates = agg("rev_rates", xi_tbl)
    a_user_self = agg("user_self", xu_tbl)
    a_sim = agg("sim", xi_tbl)
    a_has = agg("has", xi_tbl)
    a_rev_has = agg("rev_has", xa_tbl)
    a_attr_self = agg("attr_self", xa_tbl)

    hi1_ui, hi1_ui_tbl = combine(
        "ui0_rates", a_rates, cnts["rates"],
        "ui0_item_self", a_item_self, cnts["item_self"], xi, N_ITEMS, True)
    hu1, hu1_tbl = combine(
        "ui0_rev_rates", a_rev_rates, cnts["rev_rates"],
        "ui0_user_self", a_user_self, cnts["user_self"], xu, N_USERS, True)
    hi1_ii, hi1_ii_tbl = combine(
        "ii0_sim", a_sim, cnts["sim"],
        "ii0_item_self", a_item_self, cnts["item_self"], xi, N_ITEMS, True)
    ha1, ha1_tbl = combine(
        "ia0_has", a_has, cnts["has"],
        "ia0_attr_self", a_attr_self, cnts["attr_self"], xa, N_ATTRS, True)
    hi1_ia, hi1_ia_tbl = combine(
        "ia0_rev_has", a_rev_has, cnts["rev_has"],
        "ia0_item_self", a_item_self, cnts["item_self"], xi, N_ITEMS, True)

    b_rates = agg("rates", hu1_tbl)
    b_self_ui = agg("item_self", hi1_ui_tbl)
    b_sim = agg("sim", hi1_ii_tbl)
    b_self_ii = agg("item_self", hi1_ii_tbl)
    b_rev_has = agg("rev_has", ha1_tbl)
    b_self_ia = agg("item_self", hi1_ia_tbl)

    h_ui = combine("ui1_rates", b_rates, cnts["rates"],
                   "ui1_item_self", b_self_ui, cnts["item_self"],
                   hi1_ui, N_ITEMS, False)
    h_ii = combine("ii1_sim", b_sim, cnts["sim"],
                   "ii1_item_self", b_self_ii, cnts["item_self"],
                   hi1_ii, N_ITEMS, False)
    h_ia = combine("ia1_rev_has", b_rev_has, cnts["rev_has"],
                   "ia1_item_self", b_self_ia, cnts["item_self"],
                   hi1_ia, N_ITEMS, False)

    h_fused, w = _tc_fuse(p["alpha"], h_ui, h_ii, h_ia)
    return h_fused, h_ui, h_ii, h_ia, w


# speedup vs baseline: 1.0449x; 1.0449x over previous
import dataclasses
import functools

import jax
import jax.numpy as jnp
from jax import lax
from jax.experimental import pallas as pl
from jax.experimental.pallas import tpu as pltpu
from jax.experimental.pallas import tpu_sc as plsc

NC, NS = 2, 16
ACC_BIG = 50176
ACC_SMALL = 3584
ZCH = 112
N_ITEMS = 50000
N_USERS = 50000
N_ATTRS = 2000
BN = 1000


def _acc_rows(nd):
    return ACC_SMALL if nd <= ACC_SMALL - 1 else ACC_BIG


_SC_PARAMS = dataclasses.replace(pltpu.CompilerParams(),
                                 use_tc_tiling_on_sc=False)


@functools.partial(jax.jit, static_argnums=(2, 3, 4))
def _sc_agg(tables, edges, acc_rows, G, K):
    src_h, dst_h = edges
    nz = acc_rows // NS // ZCH
    rows_sub = acc_rows // NS
    mesh = plsc.VectorSubcoreMesh(core_axis_name="c", subcore_axis_name="s")

    @functools.partial(
        pl.kernel,
        out_type=jax.ShapeDtypeStruct((NC, 2, acc_rows, 64), jnp.bfloat16),
        mesh=mesh,
        scratch_types=[
            pltpu.VMEM((K, 128), jnp.int32),
            pltpu.VMEM((K, 128), jnp.int32),
            pltpu.VMEM((128, 64), jnp.bfloat16),
            pltpu.VMEM((ZCH, 64), jnp.bfloat16),
            pltpu.VMEM_SHARED((acc_rows, 64), jnp.bfloat16),
        ],
        compiler_params=_SC_PARAMS,
    )
    def body(x0, x1, zb, s_hbm, d_hbm, out_hbm,
             sidx, didx, rows, zeros, acc):
        c = lax.axis_index("c")
        s = lax.axis_index("s")
        pltpu.sync_copy(zb, zeros)
        r0 = s * rows_sub
        for q, xq in enumerate((x0, x1)):
            for j in range(nz):
                pltpu.sync_copy(zeros, acc.at[pl.ds(r0 + j * ZCH, ZCH)])
            plsc.subcore_barrier()

            @pl.loop(0, G)
            def _(g):
                pltpu.sync_copy(s_hbm.at[c, s, g], sidx)
                pltpu.sync_copy(d_hbm.at[c, s, g], didx)
                for k in range(K):
                    pltpu.sync_copy(xq.at[sidx.at[k]], rows)
                    pltpu.sync_copy(rows, acc.at[didx.at[k]], add=True)

            plsc.subcore_barrier()
            for j in range(nz):
                r = r0 + j * ZCH
                pltpu.sync_copy(acc.at[pl.ds(r, ZCH)],
                                out_hbm.at[c, q, pl.ds(r, ZCH)])
            plsc.subcore_barrier()

    zb = jnp.zeros((ZCH, 64), jnp.bfloat16)
    return body(tables[0], tables[1], zb, src_h, dst_h)


@functools.partial(jax.jit, static_argnums=(1, 2, 3))
def _sc_count(dst_h, acc_rows, G, K):
    nz = acc_rows // NS // ZCH
    rows_sub = acc_rows // NS
    mesh = plsc.VectorSubcoreMesh(core_axis_name="c", subcore_axis_name="s")

    @functools.partial(
        pl.kernel,
        out_type=jax.ShapeDtypeStruct((NC, acc_rows, 16), jnp.float32),
        mesh=mesh,
        scratch_types=[
            pltpu.VMEM((K, 128), jnp.int32),
            pltpu.VMEM((128, 16), jnp.float32),
            pltpu.VMEM((ZCH, 16), jnp.float32),
            pltpu.VMEM_SHARED((acc_rows, 16), jnp.float32),
        ],
        compiler_params=_SC_PARAMS,
    )
    def body(ones_hbm, zb, d_hbm, out_hbm, didx, ones, zeros, acc):
        c = lax.axis_index("c")
        s = lax.axis_index("s")
        pltpu.sync_copy(ones_hbm, ones)
        pltpu.sync_copy(zb, zeros)
        r0 = s * rows_sub
        for j in range(nz):
            pltpu.sync_copy(zeros, acc.at[pl.ds(r0 + j * ZCH, ZCH)])
        plsc.subcore_barrier()

        @pl.loop(0, G)
        def _(g):
            pltpu.sync_copy(d_hbm.at[c, s, g], didx)
            for k in range(K):
                pltpu.sync_copy(ones, acc.at[didx.at[k]], add=True)

        plsc.subcore_barrier()
        for j in range(nz):
            r = r0 + j * ZCH
            pltpu.sync_copy(acc.at[pl.ds(r, ZCH)], out_hbm.at[c, pl.ds(r, ZCH)])
        plsc.subcore_barrier()

    ones = jnp.ones((128, 16), jnp.float32)
    zb = jnp.zeros((ZCH, 16), jnp.float32)
    return body(ones, zb, dst_h)


def _prep_edges(ei, nd_dummy, G, K):
    e = ei.shape[1]
    epad = NC * NS * G * K * 128
    src = jnp.concatenate(
        [ei[0].astype(jnp.int32), jnp.zeros((epad - e,), jnp.int32)])
    dst = jnp.concatenate(
        [ei[1].astype(jnp.int32), jnp.full((epad - e,), nd_dummy, jnp.int32)])
    return (src.reshape(NC, NS, G, K, 128), dst.reshape(NC, NS, G, K, 128))


def _edge_plan(e):
    s = -(-e // (NC * NS * 128))
    if e >= 100000:
        s = -(-s // 8) * 8
        k = 8
    else:
        s += s % 2
        k = next(d for d in (8, 7, 6, 5, 4, 3, 2) if s % d == 0)
    return s // k, k


def _combine_body(p1, c1, p2, c2, xd, w1, w2, wr1, wr2, b1, b2, out, *tbl,
                  relu):
    inv1 = 1.0 / jnp.maximum(c1[0, :, 0] + c1[1, :, 0], 1.0)
    inv2 = 1.0 / jnp.maximum(c2[0, :, 0] + c2[1, :, 0], 1.0)
    acc = jnp.dot(xd[...], wr1[...] + wr2[...],
                  preferred_element_type=jnp.float32)
    acc += (b1[...] + b2[...])
    for h in range(2):
        m1 = (p1[0, h].astype(jnp.float32)
              + p1[1, h].astype(jnp.float32)) * inv1[:, None]
        m2 = (p2[0, h].astype(jnp.float32)
              + p2[1, h].astype(jnp.float32)) * inv2[:, None]
        acc += jnp.dot(m1, w1[h * 64:(h + 1) * 64, :],
                       preferred_element_type=jnp.float32)
        acc += jnp.dot(m2, w2[h * 64:(h + 1) * 64, :],
                       preferred_element_type=jnp.float32)
    acc *= 0.5
    o = jnp.maximum(acc, 0.0) if relu else acc
    out[...] = o
    for h in range(len(tbl)):
        tbl[h][...] = o[:, h * 64:(h + 1) * 64].astype(jnp.bfloat16)


@functools.partial(jax.jit, static_argnums=(11, 12))
def _tc_combine(p1, c1, p2, c2, xd, w1, w2, wr1, wr2, b1, b2, n, want_tbl):
    grid = (n // BN,)
    in_specs = [
        pl.BlockSpec((NC, 2, BN, 64), lambda i: (0, 0, i, 0)),
        pl.BlockSpec((NC, BN, 16), lambda i: (0, i, 0)),
        pl.BlockSpec((NC, 2, BN, 64), lambda i: (0, 0, i, 0)),
        pl.BlockSpec((NC, BN, 16), lambda i: (0, i, 0)),
        pl.BlockSpec((BN, 128), lambda i: (i, 0)),
        pl.BlockSpec((128, 128), lambda i: (0, 0)),
        pl.BlockSpec((128, 128), lambda i: (0, 0)),
        pl.BlockSpec((128, 128), lambda i: (0, 0)),
        pl.BlockSpec((128, 128), lambda i: (0, 0)),
        pl.BlockSpec((1, 128), lambda i: (0, 0)),
        pl.BlockSpec((1, 128), lambda i: (0, 0)),
    ]
    out_shape = [jax.ShapeDtypeStruct((n, 128), jnp.float32)]
    out_specs = [pl.BlockSpec((BN, 128), lambda i: (i, 0))]
    if want_tbl:
        out_shape += [jax.ShapeDtypeStruct((n, 64), jnp.bfloat16)] * 2
        out_specs += [pl.BlockSpec((BN, 64), lambda i: (i, 0))] * 2
    fn = pl.pallas_call(
        functools.partial(_combine_body, relu=True),
        grid=grid, in_specs=in_specs, out_specs=out_specs,
        out_shape=out_shape)
    res = fn(p1, c1, p2, c2, xd, w1, w2, wr1, wr2,
             b1.reshape(1, 128), b2.reshape(1, 128))
    if want_tbl:
        return res[0], res[1:]
    return res[0]


def _proj_body(x, w, b, out, *tbl):
    o = jnp.dot(x[...], w[...], preferred_element_type=jnp.float32) + b[...]
    out[...] = o
    for h in range(2):
        tbl[h][...] = o[:, h * 64:(h + 1) * 64].astype(jnp.bfloat16)


@jax.jit
def _tc_proj(x, w, b):
    n = x.shape[0]
    fn = pl.pallas_call(
        _proj_body,
        grid=(n // BN,),
        in_specs=[pl.BlockSpec((BN, 128), lambda i: (i, 0)),
                  pl.BlockSpec((128, 128), lambda i: (0, 0)),
                  pl.BlockSpec((1, 128), lambda i: (0, 0))],
        out_specs=[pl.BlockSpec((BN, 128), lambda i: (i, 0))]
        + [pl.BlockSpec((BN, 64), lambda i: (i, 0))] * 2,
        out_shape=[jax.ShapeDtypeStruct((n, 128), jnp.float32)]
        + [jax.ShapeDtypeStruct((n, 64), jnp.bfloat16)] * 2,
    )
    res = fn(x, w, b.reshape(1, 128))
    return res[0], res[1:]


def _fuse_body(apad, hui, hii, hia, out, wout):
    a = apad[...]
    m = jnp.max(a, axis=1, keepdims=True)
    e = jnp.exp(a - m)
    w = e / jnp.sum(e, axis=1, keepdims=True)
    out[...] = (hui[...] * w[0:1, 0:1] + hii[...] * w[0:1, 1:2]
                + hia[...] * w[0:1, 2:3])
    wout[...] = w


@jax.jit
def _tc_fuse(alpha, hui, hii, hia):
    n = hui.shape[0]
    apad = jnp.concatenate(
        [alpha, jnp.full((125,), -1e30, jnp.float32)]).reshape(1, 128)
    fn = pl.pallas_call(
        _fuse_body,
        grid=(n // BN,),
        in_specs=[pl.BlockSpec((1, 128), lambda i: (0, 0))]
        + [pl.BlockSpec((BN, 128), lambda i: (i, 0))] * 3,
        out_specs=[pl.BlockSpec((BN, 128), lambda i: (i, 0)),
                   pl.BlockSpec((1, 128), lambda i: (0, 0))],
        out_shape=[jax.ShapeDtypeStruct((n, 128), jnp.float32),
                   jax.ShapeDtypeStruct((1, 128), jnp.float32)],
    )
    out, wout = fn(apad, hui, hii, hia)
    return out, wout[0, :3]


def _split_cols(x):
    return tuple(x[:, h * 64:(h + 1) * 64].astype(jnp.bfloat16)
                 for h in range(2))


def kernel(x_item, ei_rates, ei_rev_rates, ei_user_self, ei_item_self,
           ei_sim, ei_has, ei_rev_has, ei_attr_self, params):
    p = params
    eis = {
        "rates": (ei_rates, N_ITEMS),
        "rev_rates": (ei_rev_rates, N_USERS),
        "user_self": (ei_user_self, N_USERS),
        "item_self": (ei_item_self, N_ITEMS),
        "sim": (ei_sim, N_ITEMS),
        "has": (ei_has, N_ATTRS),
        "rev_has": (ei_rev_has, N_ITEMS),
        "attr_self": (ei_attr_self, N_ATTRS),
    }
    prep, cnts, plans = {}, {}, {}
    for name, (ei, nd) in eis.items():
        g, k = _edge_plan(ei.shape[1])
        ar = _acc_rows(nd)
        plans[name] = (ar, g, k)
        prep[name] = _prep_edges(ei, nd, g, k)
        cnts[name] = _sc_count(prep[name][1], ar, g, k)

    def agg(name, tables):
        ar, g, k = plans[name]
        return _sc_agg(tables, prep[name], ar, g, k)

    def combine(pre1, a1, c1, pre2, a2, c2, xd, n, want_tbl):
        return _tc_combine(a1, c1, a2, c2, xd,
                           p[pre1 + "_Wl"], p[pre2 + "_Wl"],
                           p[pre1 + "_Wr"], p[pre2 + "_Wr"],
                           p[pre1 + "_bl"], p[pre2 + "_bl"], n, want_tbl)

    xi, xi_tbl = _tc_proj(x_item, p["lin_item_W"], p["lin_item_b"])
    xu = p["user_emb"]
    xu_tbl = _split_cols(xu)
    xa = p["attr_emb"]
    xa_tbl = _split_cols(xa)

    a_rates = agg("rates", xu_tbl)
    a_item_self = agg("item_self", xi_tbl)
    a_rev_rates = agg("rev_rates", xi_tbl)
    a_user_self = agg("user_self", xu_tbl)
    a_sim = agg("sim", xi_tbl)
    a_has = agg("has", xi_tbl)
    a_rev_has = agg("rev_has", xa_tbl)
    a_attr_self = agg("attr_self", xa_tbl)

    hi1_ui, hi1_ui_tbl = combine(
        "ui0_rates", a_rates, cnts["rates"],
        "ui0_item_self", a_item_self, cnts["item_self"], xi, N_ITEMS, True)
    hu1, hu1_tbl = combine(
        "ui0_rev_rates", a_rev_rates, cnts["rev_rates"],
        "ui0_user_self", a_user_self, cnts["user_self"], xu, N_USERS, True)
    hi1_ii, hi1_ii_tbl = combine(
        "ii0_sim", a_sim, cnts["sim"],
        "ii0_item_self", a_item_self, cnts["item_self"], xi, N_ITEMS, True)
    ha1, ha1_tbl = combine(
        "ia0_has", a_has, cnts["has"],
        "ia0_attr_self", a_attr_self, cnts["attr_self"], xa, N_ATTRS, True)
    hi1_ia, hi1_ia_tbl = combine(
        "ia0_rev_has", a_rev_has, cnts["rev_has"],
        "ia0_item_self", a_item_self, cnts["item_self"], xi, N_ITEMS, True)

    b_rates = agg("rates", hu1_tbl)
    b_self_ui = agg("item_self", hi1_ui_tbl)
    b_sim = agg("sim", hi1_ii_tbl)
    b_self_ii = agg("item_self", hi1_ii_tbl)
    b_rev_has = agg("rev_has", ha1_tbl)
    b_self_ia = agg("item_self", hi1_ia_tbl)

    h_ui = combine("ui1_rates", b_rates, cnts["rates"],
                   "ui1_item_self", b_self_ui, cnts["item_self"],
                   hi1_ui, N_ITEMS, False)
    h_ii = combine("ii1_sim", b_sim, cnts["sim"],
                   "ii1_item_self", b_self_ii, cnts["item_self"],
                   hi1_ii, N_ITEMS, False)
    h_ia = combine("ia1_rev_has", b_rev_has, cnts["rev_has"],
                   "ia1_item_self", b_self_ia, cnts["item_self"],
                   hi1_ia, N_ITEMS, False)

    h_fused, w = _tc_fuse(p["alpha"], h_ui, h_ii, h_ia)
    return h_fused, h_ui, h_ii, h_ia, w


# speedup vs baseline: 1.0642x; 1.0184x over previous
import dataclasses
import functools

import jax
import jax.numpy as jnp
from jax import lax
from jax.experimental import pallas as pl
from jax.experimental.pallas import tpu as pltpu
from jax.experimental.pallas import tpu_sc as plsc

NC, NS = 2, 16
ACC_BIG = 50176
ACC_SMALL = 3584
ZCH = 112
N_ITEMS = 50000
N_USERS = 50000
N_ATTRS = 2000
BN = 1000


def _acc_rows(nd):
    return ACC_SMALL if nd <= ACC_SMALL - 1 else ACC_BIG


_SC_PARAMS = dataclasses.replace(pltpu.CompilerParams(),
                                 use_tc_tiling_on_sc=False)


@functools.partial(jax.jit, static_argnums=(2, 3, 4))
def _sc_agg(tables, edges, acc_rows, G, K):
    src_h, dst_h = edges
    nz = acc_rows // NS // ZCH
    rows_sub = acc_rows // NS
    mesh = plsc.VectorSubcoreMesh(core_axis_name="c", subcore_axis_name="s")

    @functools.partial(
        pl.kernel,
        out_type=jax.ShapeDtypeStruct((NC, 2, acc_rows, 64), jnp.bfloat16),
        mesh=mesh,
        scratch_types=[
            pltpu.VMEM((K, 128), jnp.int32),
            pltpu.VMEM((K, 128), jnp.int32),
            pltpu.VMEM((128, 64), jnp.bfloat16),
            pltpu.VMEM((ZCH, 64), jnp.bfloat16),
            pltpu.VMEM_SHARED((acc_rows, 64), jnp.bfloat16),
        ],
        compiler_params=_SC_PARAMS,
    )
    def body(x0, x1, zb, s_hbm, d_hbm, out_hbm,
             sidx, didx, rows, zeros, acc):
        c = lax.axis_index("c")
        s = lax.axis_index("s")
        pltpu.sync_copy(zb, zeros)
        r0 = s * rows_sub
        for q, xq in enumerate((x0, x1)):
            for j in range(nz):
                pltpu.sync_copy(zeros, acc.at[pl.ds(r0 + j * ZCH, ZCH)])
            plsc.subcore_barrier()

            @pl.loop(0, G)
            def _(g):
                pltpu.sync_copy(s_hbm.at[c, s, g], sidx)
                pltpu.sync_copy(d_hbm.at[c, s, g], didx)
                for k in range(K):
                    pltpu.sync_copy(xq.at[sidx.at[k]], rows)
                    pltpu.sync_copy(rows, acc.at[didx.at[k]], add=True)

            plsc.subcore_barrier()
            for j in range(nz):
                r = r0 + j * ZCH
                pltpu.sync_copy(acc.at[pl.ds(r, ZCH)],
                                out_hbm.at[c, q, pl.ds(r, ZCH)])
            plsc.subcore_barrier()

    zb = jnp.zeros((ZCH, 64), jnp.bfloat16)
    return body(tables[0], tables[1], zb, src_h, dst_h)


@functools.partial(jax.jit, static_argnums=(1,))
def _sc_count_all(dst_list, plans):
    mesh = plsc.VectorSubcoreMesh(core_axis_name="c", subcore_axis_name="s")
    kmax = max(k for _, _, k in plans)
    armax = max(ar for ar, _, _ in plans)

    @functools.partial(
        pl.kernel,
        out_type=[jax.ShapeDtypeStruct((NC, ar, 16), jnp.float32)
                  for ar, _, _ in plans],
        mesh=mesh,
        scratch_types=[
            pltpu.VMEM((kmax, 128), jnp.int32),
            pltpu.VMEM((128, 16), jnp.float32),
            pltpu.VMEM((ZCH, 16), jnp.float32),
            pltpu.VMEM_SHARED((armax, 16), jnp.float32),
        ],
        compiler_params=_SC_PARAMS,
    )
    def body(ones_hbm, zb, *rest):
        n = len(plans)
        d_hbms, out_hbms = rest[:n], rest[n:2 * n]
        didx, ones, zeros, acc = rest[2 * n:]
        c = lax.axis_index("c")
        s = lax.axis_index("s")
        pltpu.sync_copy(ones_hbm, ones)
        pltpu.sync_copy(zb, zeros)
        for (ar, grp, kk), d_hbm, out_hbm in zip(plans, d_hbms, out_hbms):
            nz = ar // NS // ZCH
            rows_sub = ar // NS
            r0 = s * rows_sub
            for j in range(nz):
                pltpu.sync_copy(zeros, acc.at[pl.ds(r0 + j * ZCH, ZCH)])
            plsc.subcore_barrier()

            @pl.loop(0, grp)
            def _(g):
                pltpu.sync_copy(d_hbm.at[c, s, g], didx.at[pl.ds(0, kk)])
                for k in range(kk):
                    pltpu.sync_copy(ones, acc.at[didx.at[k]], add=True)

            plsc.subcore_barrier()
            for j in range(nz):
                r = r0 + j * ZCH
                pltpu.sync_copy(acc.at[pl.ds(r, ZCH)],
                                out_hbm.at[c, pl.ds(r, ZCH)])
            plsc.subcore_barrier()

    ones = jnp.ones((128, 16), jnp.float32)
    zb = jnp.zeros((ZCH, 16), jnp.float32)
    return body(ones, zb, *dst_list)


def _prep_edges(ei, nd_dummy, G, K):
    e = ei.shape[1]
    epad = NC * NS * G * K * 128
    src = jnp.concatenate(
        [ei[0].astype(jnp.int32), jnp.zeros((epad - e,), jnp.int32)])
    dst = jnp.concatenate(
        [ei[1].astype(jnp.int32), jnp.full((epad - e,), nd_dummy, jnp.int32)])
    return (src.reshape(NC, NS, G, K, 128), dst.reshape(NC, NS, G, K, 128))


def _edge_plan(e):
    s = -(-e // (NC * NS * 128))
    if e >= 100000:
        s = -(-s // 8) * 8
        k = 8
    else:
        s += s % 2
        k = next(d for d in (8, 7, 6, 5, 4, 3, 2) if s % d == 0)
    return s // k, k


def _combine_body(p1, c1, p2, c2, xd, w1, w2, wr1, wr2, b1, b2, out, *tbl,
                  relu):
    inv1 = 1.0 / jnp.maximum(c1[0, :, 0] + c1[1, :, 0], 1.0)
    inv2 = 1.0 / jnp.maximum(c2[0, :, 0] + c2[1, :, 0], 1.0)
    acc = jnp.dot(xd[...], wr1[...] + wr2[...],
                  preferred_element_type=jnp.float32)
    acc += (b1[...] + b2[...])
    for h in range(2):
        m1 = (p1[0, h].astype(jnp.float32)
              + p1[1, h].astype(jnp.float32)) * inv1[:, None]
        m2 = (p2[0, h].astype(jnp.float32)
              + p2[1, h].astype(jnp.float32)) * inv2[:, None]
        acc += jnp.dot(m1, w1[h * 64:(h + 1) * 64, :],
                       preferred_element_type=jnp.float32)
        acc += jnp.dot(m2, w2[h * 64:(h + 1) * 64, :],
                       preferred_element_type=jnp.float32)
    acc *= 0.5
    o = jnp.maximum(acc, 0.0) if relu else acc
    out[...] = o
    for h in range(len(tbl)):
        tbl[h][...] = o[:, h * 64:(h + 1) * 64].astype(jnp.bfloat16)


@functools.partial(jax.jit, static_argnums=(11, 12))
def _tc_combine(p1, c1, p2, c2, xd, w1, w2, wr1, wr2, b1, b2, n, want_tbl):
    grid = (n // BN,)
    in_specs = [
        pl.BlockSpec((NC, 2, BN, 64), lambda i: (0, 0, i, 0)),
        pl.BlockSpec((NC, BN, 16), lambda i: (0, i, 0)),
        pl.BlockSpec((NC, 2, BN, 64), lambda i: (0, 0, i, 0)),
        pl.BlockSpec((NC, BN, 16), lambda i: (0, i, 0)),
        pl.BlockSpec((BN, 128), lambda i: (i, 0)),
        pl.BlockSpec((128, 128), lambda i: (0, 0)),
        pl.BlockSpec((128, 128), lambda i: (0, 0)),
        pl.BlockSpec((128, 128), lambda i: (0, 0)),
        pl.BlockSpec((128, 128), lambda i: (0, 0)),
        pl.BlockSpec((1, 128), lambda i: (0, 0)),
        pl.BlockSpec((1, 128), lambda i: (0, 0)),
    ]
    out_shape = [jax.ShapeDtypeStruct((n, 128), jnp.float32)]
    out_specs = [pl.BlockSpec((BN, 128), lambda i: (i, 0))]
    if want_tbl:
        out_shape += [jax.ShapeDtypeStruct((n, 64), jnp.bfloat16)] * 2
        out_specs += [pl.BlockSpec((BN, 64), lambda i: (i, 0))] * 2
    fn = pl.pallas_call(
        functools.partial(_combine_body, relu=True),
        grid=grid, in_specs=in_specs, out_specs=out_specs,
        out_shape=out_shape)
    res = fn(p1, c1, p2, c2, xd, w1, w2, wr1, wr2,
             b1.reshape(1, 128), b2.reshape(1, 128))
    if want_tbl:
        return res[0], res[1:]
    return res[0]


def _proj_body(x, w, b, out, *tbl):
    o = jnp.dot(x[...], w[...], preferred_element_type=jnp.float32) + b[...]
    out[...] = o
    for h in range(2):
        tbl[h][...] = o[:, h * 64:(h + 1) * 64].astype(jnp.bfloat16)


@jax.jit
def _tc_proj(x, w, b):
    n = x.shape[0]
    fn = pl.pallas_call(
        _proj_body,
        grid=(n // BN,),
        in_specs=[pl.BlockSpec((BN, 128), lambda i: (i, 0)),
                  pl.BlockSpec((128, 128), lambda i: (0, 0)),
                  pl.BlockSpec((1, 128), lambda i: (0, 0))],
        out_specs=[pl.BlockSpec((BN, 128), lambda i: (i, 0))]
        + [pl.BlockSpec((BN, 64), lambda i: (i, 0))] * 2,
        out_shape=[jax.ShapeDtypeStruct((n, 128), jnp.float32)]
        + [jax.ShapeDtypeStruct((n, 64), jnp.bfloat16)] * 2,
    )
    res = fn(x, w, b.reshape(1, 128))
    return res[0], res[1:]


def _fuse_body(apad, hui, hii, hia, out, wout):
    a = apad[...]
    m = jnp.max(a, axis=1, keepdims=True)
    e = jnp.exp(a - m)
    w = e / jnp.sum(e, axis=1, keepdims=True)
    out[...] = (hui[...] * w[0:1, 0:1] + hii[...] * w[0:1, 1:2]
                + hia[...] * w[0:1, 2:3])
    wout[...] = w


@jax.jit
def _tc_fuse(alpha, hui, hii, hia):
    n = hui.shape[0]
    apad = jnp.concatenate(
        [alpha, jnp.full((125,), -1e30, jnp.float32)]).reshape(1, 128)
    fn = pl.pallas_call(
        _fuse_body,
        grid=(n // BN,),
        in_specs=[pl.BlockSpec((1, 128), lambda i: (0, 0))]
        + [pl.BlockSpec((BN, 128), lambda i: (i, 0))] * 3,
        out_specs=[pl.BlockSpec((BN, 128), lambda i: (i, 0)),
                   pl.BlockSpec((1, 128), lambda i: (0, 0))],
        out_shape=[jax.ShapeDtypeStruct((n, 128), jnp.float32),
                   jax.ShapeDtypeStruct((1, 128), jnp.float32)],
    )
    out, wout = fn(apad, hui, hii, hia)
    return out, wout[0, :3]


def _split_cols(x):
    return tuple(x[:, h * 64:(h + 1) * 64].astype(jnp.bfloat16)
                 for h in range(2))


def kernel(x_item, ei_rates, ei_rev_rates, ei_user_self, ei_item_self,
           ei_sim, ei_has, ei_rev_has, ei_attr_self, params):
    p = params
    eis = {
        "rates": (ei_rates, N_ITEMS),
        "rev_rates": (ei_rev_rates, N_USERS),
        "user_self": (ei_user_self, N_USERS),
        "item_self": (ei_item_self, N_ITEMS),
        "sim": (ei_sim, N_ITEMS),
        "has": (ei_has, N_ATTRS),
        "rev_has": (ei_rev_has, N_ITEMS),
        "attr_self": (ei_attr_self, N_ATTRS),
    }
    prep, plans = {}, {}
    for name, (ei, nd) in eis.items():
        g, k = _edge_plan(ei.shape[1])
        ar = _acc_rows(nd)
        plans[name] = (ar, g, k)
        prep[name] = _prep_edges(ei, nd, g, k)
    names = list(eis)
    cnt_list = _sc_count_all([prep[n][1] for n in names],
                             tuple(plans[n] for n in names))
    cnts = dict(zip(names, cnt_list))

    def agg(name, tables):
        ar, g, k = plans[name]
        return _sc_agg(tables, prep[name], ar, g, k)

    def combine(pre1, a1, c1, pre2, a2, c2, xd, n, want_tbl):
        return _tc_combine(a1, c1, a2, c2, xd,
                           p[pre1 + "_Wl"], p[pre2 + "_Wl"],
                           p[pre1 + "_Wr"], p[pre2 + "_Wr"],
                           p[pre1 + "_bl"], p[pre2 + "_bl"], n, want_tbl)

    xi, xi_tbl = _tc_proj(x_item, p["lin_item_W"], p["lin_item_b"])
    xu = p["user_emb"]
    xu_tbl = _split_cols(xu)
    xa = p["attr_emb"]
    xa_tbl = _split_cols(xa)

    a_rates = agg("rates", xu_tbl)
    a_item_self = agg("item_self", xi_tbl)
    a_rev_rates = agg("rev_rates", xi_tbl)
    a_user_self = agg("user_self", xu_tbl)
    a_sim = agg("sim", xi_tbl)
    a_has = agg("has", xi_tbl)
    a_rev_has = agg("rev_has", xa_tbl)
    a_attr_self = agg("attr_self", xa_tbl)

    hi1_ui, hi1_ui_tbl = combine(
        "ui0_rates", a_rates, cnts["rates"],
        "ui0_item_self", a_item_self, cnts["item_self"], xi, N_ITEMS, True)
    hu1, hu1_tbl = combine(
        "ui0_rev_rates", a_rev_rates, cnts["rev_rates"],
        "ui0_user_self", a_user_self, cnts["user_self"], xu, N_USERS, True)
    hi1_ii, hi1_ii_tbl = combine(
        "ii0_sim", a_sim, cnts["sim"],
        "ii0_item_self", a_item_self, cnts["item_self"], xi, N_ITEMS, True)
    ha1, ha1_tbl = combine(
        "ia0_has", a_has, cnts["has"],
        "ia0_attr_self", a_attr_self, cnts["attr_self"], xa, N_ATTRS, True)
    hi1_ia, hi1_ia_tbl = combine(
        "ia0_rev_has", a_rev_has, cnts["rev_has"],
        "ia0_item_self", a_item_self, cnts["item_self"], xi, N_ITEMS, True)

    b_rates = agg("rates", hu1_tbl)
    b_self_ui = agg("item_self", hi1_ui_tbl)
    b_sim = agg("sim", hi1_ii_tbl)
    b_self_ii = agg("item_self", hi1_ii_tbl)
    b_rev_has = agg("rev_has", ha1_tbl)
    b_self_ia = agg("item_self", hi1_ia_tbl)

    h_ui = combine("ui1_rates", b_rates, cnts["rates"],
                   "ui1_item_self", b_self_ui, cnts["item_self"],
                   hi1_ui, N_ITEMS, False)
    h_ii = combine("ii1_sim", b_sim, cnts["sim"],
                   "ii1_item_self", b_self_ii, cnts["item_self"],
                   hi1_ii, N_ITEMS, False)
    h_ia = combine("ia1_rev_has", b_rev_has, cnts["rev_has"],
                   "ia1_item_self", b_self_ia, cnts["item_self"],
                   hi1_ia, N_ITEMS, False)

    h_fused, w = _tc_fuse(p["alpha"], h_ui, h_ii, h_ia)
    return h_fused, h_ui, h_ii, h_ia, w
